# Initial kernel scaffold; baseline (speedup 1.0000x reference)
import functools

import jax
import jax.numpy as jnp
from jax import lax
from jax.experimental import pallas as pl
from jax.experimental.pallas import tpu as pltpu
from jax.experimental.pallas import tpu_sc as plsc

_N_CLS = 1024
_SEQ = 77
_N_CTX = 16
_CTX_DIM = 512
_NUM_CORES = 2
_NUM_SUBCORES = 16
_NW = _NUM_CORES * _NUM_SUBCORES
_CPW = _N_CLS // _NW
_IDX_PAD = 72


def _assemble_body(idx_hbm, ctx_hbm, table_hbm, out_hbm,
                   idx_v, staging_v, gsem):
    wid = lax.axis_index("s") * _NUM_CORES + lax.axis_index("c")
    pltpu.sync_copy(ctx_hbm, staging_v.at[pl.ds(1, _N_CTX)])

    def body(i, carry):
        c = wid * _CPW + i
        pltpu.sync_copy(idx_hbm.at[c], idx_v)
        sfx = pltpu.async_copy(
            table_hbm.at[idx_v.at[pl.ds(0, 60)]],
            staging_v.at[pl.ds(1 + _N_CTX, 60)], gsem)
        sos = pltpu.async_copy(
            table_hbm.at[idx_v.at[pl.ds(64, 1)]],
            staging_v.at[pl.ds(0, 1)], gsem)
        sfx.wait()
        sos.wait()
        pltpu.sync_copy(staging_v, out_hbm.at[c])
        return carry

    lax.fori_loop(0, _CPW, body, 0)


@jax.jit
def _assemble(idx, ctx, token_embedding):
    mesh = plsc.VectorSubcoreMesh(
        core_axis_name="c", subcore_axis_name="s",
        num_cores=_NUM_CORES, num_subcores=_NUM_SUBCORES)
    return pl.kernel(
        _assemble_body,
        out_type=jax.ShapeDtypeStruct((_N_CLS, _SEQ, _CTX_DIM), jnp.float32),
        mesh=mesh,
        scratch_types=[
            pltpu.VMEM((_IDX_PAD,), jnp.int32),
            pltpu.VMEM((_SEQ, _CTX_DIM), jnp.float32),
            pltpu.SemaphoreType.DMA,
        ],
    )(idx, ctx, token_embedding)


def kernel(tokenized_prompts, ctx, token_embedding):
    idx = jnp.zeros((_N_CLS, _IDX_PAD), jnp.int32)
    idx = idx.at[:, :_SEQ - 1 - _N_CTX].set(tokenized_prompts[:, 1 + _N_CTX:])
    idx = idx.at[:, 64].set(tokenized_prompts[:, 0])
    prompts = _assemble(idx, ctx, token_embedding)
    return prompts, tokenized_prompts


# speedup vs baseline: 1.4226x; 1.4226x over previous
import functools

import jax
import jax.numpy as jnp
from jax import lax
from jax.experimental import pallas as pl
from jax.experimental.pallas import tpu as pltpu
from jax.experimental.pallas import tpu_sc as plsc

_N_CLS = 1024
_SEQ = 77
_N_CTX = 16
_CTX_DIM = 512
_NUM_CORES = 2
_NUM_SUBCORES = 16
_NW = _NUM_CORES * _NUM_SUBCORES
_CPW = _N_CLS // _NW
_IDX_PAD = 72


def _assemble_body(idx_hbm, ctx_hbm, table_hbm, out_hbm,
                   idx_v, staging_v, gsem):
    wid = lax.axis_index("s") * _NUM_CORES + lax.axis_index("c")
    pltpu.sync_copy(ctx_hbm, staging_v.at[pl.ds(1, _N_CTX)])

    def body(i, carry):
        c = wid * _CPW + i
        pltpu.sync_copy(idx_hbm.at[c], idx_v)
        sfx = pltpu.async_copy(
            table_hbm.at[idx_v.at[pl.ds(0, 60)]],
            staging_v.at[pl.ds(1 + _N_CTX, 60)], gsem)
        sos = pltpu.async_copy(
            table_hbm.at[idx_v.at[pl.ds(64, 1)]],
            staging_v.at[pl.ds(0, 1)], gsem)
        sfx.wait()
        sos.wait()
        pltpu.sync_copy(staging_v, out_hbm.at[c])
        return carry

    lax.fori_loop(0, _CPW, body, 0)


@jax.jit
def _assemble(idx, ctx, token_embedding):
    mesh = plsc.VectorSubcoreMesh(
        core_axis_name="c", subcore_axis_name="s",
        num_cores=_NUM_CORES, num_subcores=_NUM_SUBCORES)
    return pl.kernel(
        _assemble_body,
        out_type=jax.ShapeDtypeStruct((_N_CLS, _SEQ, _CTX_DIM), jnp.float32),
        mesh=mesh,
        scratch_types=[
            pltpu.VMEM((_IDX_PAD,), jnp.int32),
            pltpu.VMEM((_SEQ, _CTX_DIM), jnp.float32),
            pltpu.SemaphoreType.DMA,
        ],
        compiler_params=pltpu.CompilerParams(use_tc_tiling_on_sc=False),
    )(idx, ctx, token_embedding)


def kernel(tokenized_prompts, ctx, token_embedding):
    idx = jnp.zeros((_N_CLS, _IDX_PAD), jnp.int32)
    idx = idx.at[:, :_SEQ - 1 - _N_CTX].set(tokenized_prompts[:, 1 + _N_CTX:])
    idx = idx.at[:, 64].set(tokenized_prompts[:, 0])
    prompts = _assemble(idx, ctx, token_embedding)
    return prompts, tokenized_prompts


# speedup vs baseline: 1.5315x; 1.0765x over previous
import functools

import jax
import jax.numpy as jnp
from jax import lax
from jax.experimental import pallas as pl
from jax.experimental.pallas import tpu as pltpu
from jax.experimental.pallas import tpu_sc as plsc

_N_CLS = 1024
_SEQ = 77
_N_CTX = 16
_CTX_DIM = 512
_NUM_CORES = 2
_NUM_SUBCORES = 16
_NW = _NUM_CORES * _NUM_SUBCORES
_CPW = _N_CLS // _NW
_IDX_PAD = 72


def _assemble_body(idx_hbm, ctx_hbm, table_hbm, out_hbm,
                   idx_v, staging_v, gsem, ssem):
    wid = lax.axis_index("s") * _NUM_CORES + lax.axis_index("c")
    pltpu.sync_copy(
        idx_hbm.at[pl.ds(wid * _CPW * _IDX_PAD, _CPW * _IDX_PAD)], idx_v)
    pltpu.sync_copy(ctx_hbm, staging_v.at[0, pl.ds(1, _N_CTX)])
    pltpu.sync_copy(ctx_hbm, staging_v.at[1, pl.ds(1, _N_CTX)])

    def body(i, carry):
        c = wid * _CPW + i
        b = lax.rem(i, 2)
        @pl.when(i >= 2)
        def _():
            pltpu.make_async_copy(
                staging_v.at[b], out_hbm.at[c], ssem.at[b]).wait()
        sfx = pltpu.async_copy(
            table_hbm.at[idx_v.at[pl.ds(i * _IDX_PAD, 60)]],
            staging_v.at[b, pl.ds(1 + _N_CTX, 60)], gsem)
        sos = pltpu.async_copy(
            table_hbm.at[idx_v.at[pl.ds(i * _IDX_PAD + 64, 1)]],
            staging_v.at[b, pl.ds(0, 1)], gsem)
        sfx.wait()
        sos.wait()
        pltpu.async_copy(staging_v.at[b], out_hbm.at[c], ssem.at[b])
        return carry

    lax.fori_loop(0, _CPW, body, 0)
    pltpu.make_async_copy(staging_v.at[0], out_hbm.at[0], ssem.at[0]).wait()
    pltpu.make_async_copy(staging_v.at[1], out_hbm.at[0], ssem.at[1]).wait()


@jax.jit
def _assemble(idx, ctx, token_embedding):
    mesh = plsc.VectorSubcoreMesh(
        core_axis_name="c", subcore_axis_name="s",
        num_cores=_NUM_CORES, num_subcores=_NUM_SUBCORES)
    return pl.kernel(
        _assemble_body,
        out_type=jax.ShapeDtypeStruct((_N_CLS, _SEQ, _CTX_DIM), jnp.float32),
        mesh=mesh,
        scratch_types=[
            pltpu.VMEM((_CPW * _IDX_PAD,), jnp.int32),
            pltpu.VMEM((2, _SEQ, _CTX_DIM), jnp.float32),
            pltpu.SemaphoreType.DMA,
            pltpu.SemaphoreType.DMA((2,)),
        ],
        compiler_params=pltpu.CompilerParams(use_tc_tiling_on_sc=False),
    )(idx, ctx, token_embedding)


def kernel(tokenized_prompts, ctx, token_embedding):
    idx = jnp.zeros((_N_CLS, _IDX_PAD), jnp.int32)
    idx = idx.at[:, :_SEQ - 1 - _N_CTX].set(tokenized_prompts[:, 1 + _N_CTX:])
    idx = idx.at[:, 64].set(tokenized_prompts[:, 0])
    prompts = _assemble(idx.reshape(-1), ctx, token_embedding)
    return prompts, tokenized_prompts


# speedup vs baseline: 5.2279x; 3.4137x over previous
import functools

import jax
import jax.numpy as jnp
from jax import lax
from jax.experimental import pallas as pl
from jax.experimental.pallas import tpu as pltpu
from jax.experimental.pallas import tpu_sc as plsc

_N_CLS = 1024
_SEQ = 77
_N_CTX = 16
_CTX_DIM = 512
_NUM_CORES = 2
_NUM_SUBCORES = 16
_NW = _NUM_CORES * _NUM_SUBCORES
_CHUNK = 64
_NCHUNK = _N_CLS // _CHUNK
_NGU = (_SEQ - _N_CTX) * _NCHUNK
_BASE = 2 * (_NGU // (2 * _NW))


def _unit_coords(u):
    q = u // _NCHUNK
    t = jnp.where(q == 0, 0, q + _N_CTX)
    c0 = (u % _NCHUNK) * _CHUNK
    return t, c0


def _assemble_body(tokt_hbm, ctx64_hbm, table_hbm, out_hbm,
                   stage0, stage1, ctx_rep, idx0, idx1,
                   gsem, ssem0, ssem1, csem):
    wid = lax.axis_index("s") * _NUM_CORES + lax.axis_index("c")

    tctx = wid // 2
    cbase = (wid % 2) * (_NCHUNK // 2) * _CHUNK
    pltpu.sync_copy(ctx64_hbm.at[pl.ds(tctx * _CHUNK, _CHUNK)], ctx_rep)
    for k in range(_NCHUNK // 2):
        pltpu.async_copy(
            ctx_rep, out_hbm.at[tctx + 1, pl.ds(cbase + k * _CHUNK, _CHUNK)],
            csem)

    def _gather(u, idx_v, stage_v):
        t, c0 = _unit_coords(u)
        pltpu.sync_copy(tokt_hbm.at[pl.ds(t * _N_CLS + c0, _CHUNK)], idx_v)
        pltpu.async_copy(table_hbm.at[idx_v], stage_v, gsem).wait()
        return t, c0

    def _store(stage_v, t, c0, sem):
        pltpu.async_copy(stage_v, out_hbm.at[t, pl.ds(c0, _CHUNK)], sem)

    def body(i, carry):
        u_a = wid + (2 * i) * _NW
        u_b = u_a + _NW

        @pl.when(i > 0)
        def _():
            pltpu.make_async_copy(
                stage0, out_hbm.at[0, pl.ds(0, _CHUNK)], ssem0).wait()
        t_a, c_a = _gather(u_a, idx0, stage0)
        _store(stage0, t_a, c_a, ssem0)

        @pl.when(i > 0)
        def _():
            pltpu.make_async_copy(
                stage1, out_hbm.at[0, pl.ds(0, _CHUNK)], ssem1).wait()
        t_b, c_b = _gather(u_b, idx1, stage1)
        _store(stage1, t_b, c_b, ssem1)
        return carry

    lax.fori_loop(0, _BASE // 2, body, 0)
    pltpu.make_async_copy(stage0, out_hbm.at[0, pl.ds(0, _CHUNK)], ssem0).wait()
    pltpu.make_async_copy(stage1, out_hbm.at[0, pl.ds(0, _CHUNK)], ssem1).wait()

    @pl.when(wid < _NGU - _BASE * _NW)
    def _():
        u = _BASE * _NW + wid
        t, c0 = _gather(u, idx0, stage0)
        pltpu.async_copy(stage0, out_hbm.at[t, pl.ds(c0, _CHUNK)], ssem0).wait()

    for _k in range(_NCHUNK // 2):
        pltpu.make_async_copy(
            ctx_rep, out_hbm.at[0, pl.ds(0, _CHUNK)], csem).wait()


@jax.jit
def _assemble(tokt, ctx64, token_embedding):
    mesh = plsc.VectorSubcoreMesh(
        core_axis_name="c", subcore_axis_name="s",
        num_cores=_NUM_CORES, num_subcores=_NUM_SUBCORES)
    return pl.kernel(
        _assemble_body,
        out_type=jax.ShapeDtypeStruct((_SEQ, _N_CLS, _CTX_DIM), jnp.float32),
        mesh=mesh,
        scratch_types=[
            pltpu.VMEM((_CHUNK, _CTX_DIM), jnp.float32),
            pltpu.VMEM((_CHUNK, _CTX_DIM), jnp.float32),
            pltpu.VMEM((_CHUNK, _CTX_DIM), jnp.float32),
            pltpu.VMEM((_CHUNK,), jnp.int32),
            pltpu.VMEM((_CHUNK,), jnp.int32),
            pltpu.SemaphoreType.DMA,
            pltpu.SemaphoreType.DMA,
            pltpu.SemaphoreType.DMA,
            pltpu.SemaphoreType.DMA,
        ],
        compiler_params=pltpu.CompilerParams(use_tc_tiling_on_sc=True),
    )(tokt, ctx64, token_embedding)


def kernel(tokenized_prompts, ctx, token_embedding):
    tokt = tokenized_prompts.T.reshape(-1)
    ctx64 = jnp.repeat(ctx, _CHUNK, axis=0)
    out = _assemble(tokt, ctx64, token_embedding)
    return jnp.transpose(out, (1, 0, 2)), tokenized_prompts


# speedup vs baseline: 5.6231x; 1.0756x over previous
import functools

import jax
import jax.numpy as jnp
from jax import lax
from jax.experimental import pallas as pl
from jax.experimental.pallas import tpu as pltpu
from jax.experimental.pallas import tpu_sc as plsc

_N_CLS = 1024
_SEQ = 77
_N_CTX = 16
_CTX_DIM = 512
_NUM_CORES = 2
_NUM_SUBCORES = 16
_NW = _NUM_CORES * _NUM_SUBCORES
_CHUNK = 64
_NCHUNK = _N_CLS // _CHUNK
_NPOS = _SEQ - _N_CTX
_NGU = _NPOS * _NCHUNK
_NU_HI = -(-_NGU // _NW)
_NU_LO = _NGU // _NW
_N_HI = _NGU - _NU_LO * _NW
_CTX_ROWS = 32


def _assemble_body(tokg_hbm, ctx64_hbm, table_hbm, out_hbm,
                   st0, st1, st2, ctx_rep, idx_all,
                   g0, g1, g2, s0, s1, s2, csem):
    wid = lax.axis_index("s") * _NUM_CORES + lax.axis_index("c")
    is_hi = wid < _N_HI
    nu = jnp.where(is_hi, _NU_HI, _NU_LO)
    base = jnp.where(is_hi, _NU_HI * wid,
                     _NU_HI * _N_HI + _NU_LO * (wid - _N_HI))

    stages = (st0, st1, st2)
    gsems = (g0, g1, g2)
    ssems = (s0, s1, s2)

    def unit_out(u):
        q = u // _NCHUNK
        t = jnp.where(q == 0, 0, q + _N_CTX)
        c0 = (u % _NCHUNK) * _CHUNK
        return t, c0

    def fire_gather(i, k):
        pltpu.async_copy(
            table_hbm.at[idx_all.at[pl.ds(i * _CHUNK, _CHUNK)]],
            stages[k], gsems[k])

    def wait_gather(k):
        pltpu.make_async_copy(
            table_hbm.at[pl.ds(0, _CHUNK)], stages[k], gsems[k]).wait()

    def fire_store(i, k):
        t, c0 = unit_out(base + i)
        pltpu.async_copy(
            stages[k], out_hbm.at[t, pl.ds(c0, _CHUNK)], ssems[k])

    def wait_store(k):
        pltpu.make_async_copy(
            stages[k], out_hbm.at[0, pl.ds(0, _CHUNK)], ssems[k]).wait()

    pltpu.sync_copy(
        tokg_hbm.at[pl.ds(base * _CHUNK, _NU_HI * _CHUNK)], idx_all)
    fire_gather(0, 0)

    tctx = wid // 2
    cbase = (wid % 2) * (_N_CLS // 2)
    pltpu.sync_copy(
        ctx64_hbm.at[pl.ds(tctx * _CHUNK, _CTX_ROWS)], ctx_rep)
    for k in range(_N_CLS // 2 // _CTX_ROWS):
        pltpu.async_copy(
            ctx_rep,
            out_hbm.at[tctx + 1, pl.ds(cbase + k * _CTX_ROWS, _CTX_ROWS)],
            csem)

    def body(ip, carry):
        for k in range(3):
            u = 3 * ip + k

            @pl.when(u + 1 < nu)
            def _(u=u, k=k):
                if k == 2:
                    wait_store((k + 1) % 3)
                else:
                    @pl.when(ip > 0)
                    def _():
                        wait_store((k + 1) % 3)
                fire_gather(u + 1, (k + 1) % 3)

            @pl.when(u < nu)
            def _(u=u, k=k):
                wait_gather(k)
                fire_store(u, k)
        return carry

    lax.fori_loop(0, -(-_NU_HI // 3), body, 0)
    wait_store(0)
    wait_store(1)
    wait_store(2)
    for _k in range(_N_CLS // 2 // _CTX_ROWS):
        pltpu.make_async_copy(
            ctx_rep, out_hbm.at[0, pl.ds(0, _CTX_ROWS)], csem).wait()


@jax.jit
def _assemble(tokg, ctx64, token_embedding):
    mesh = plsc.VectorSubcoreMesh(
        core_axis_name="c", subcore_axis_name="s",
        num_cores=_NUM_CORES, num_subcores=_NUM_SUBCORES)
    return pl.kernel(
        _assemble_body,
        out_type=jax.ShapeDtypeStruct((_SEQ, _N_CLS, _CTX_DIM), jnp.float32),
        mesh=mesh,
        scratch_types=[
            pltpu.VMEM((_CHUNK, _CTX_DIM), jnp.float32),
            pltpu.VMEM((_CHUNK, _CTX_DIM), jnp.float32),
            pltpu.VMEM((_CHUNK, _CTX_DIM), jnp.float32),
            pltpu.VMEM((_CTX_ROWS, _CTX_DIM), jnp.float32),
            pltpu.VMEM((_NU_HI * _CHUNK,), jnp.int32),
            pltpu.SemaphoreType.DMA,
            pltpu.SemaphoreType.DMA,
            pltpu.SemaphoreType.DMA,
            pltpu.SemaphoreType.DMA,
            pltpu.SemaphoreType.DMA,
            pltpu.SemaphoreType.DMA,
            pltpu.SemaphoreType.DMA,
        ],
        compiler_params=pltpu.CompilerParams(use_tc_tiling_on_sc=True),
    )(tokg, ctx64, token_embedding)


def kernel(tokenized_prompts, ctx, token_embedding):
    tokg = jnp.concatenate(
        [tokenized_prompts[:, :1], tokenized_prompts[:, 1 + _N_CTX:]], axis=1)
    tokg = tokg.T.reshape(-1)
    tokg = jnp.concatenate([tokg, jnp.zeros((_NU_HI * _CHUNK,), jnp.int32)])
    ctx64 = jnp.repeat(ctx, _CHUNK, axis=0)
    out = _assemble(tokg, ctx64, token_embedding)
    return jnp.transpose(out, (1, 0, 2)), tokenized_prompts
